# baseline (speedup 1.0000x reference)
import jax
import jax.numpy as jnp
from jax import lax
from jax.experimental import pallas as pl
from jax.experimental.pallas import tpu as pltpu
from jax.experimental.pallas import tpu_sc as plsc

NB_EMBEDDINGS = 100000
NROWS = 2 * NB_EMBEDDINGS
EMB_DIM = 64
BATCH = 16384

NUM_CORES = 2
NUM_SUBCORES = 16
LANES = 16
NUM_WORKERS = NUM_CORES * NUM_SUBCORES
BPW = BATCH // NUM_WORKERS
CHUNK = 128
NCHUNKS = BPW // CHUNK
GROUPS = BPW // LANES


def _sc_body(pair_hbm, table_hbm, bias_hbm, out_hbm,
             pairs_v, idx0_v, idx1_v, rows0_v, rows1_v, b0_v, b1_v, out_v,
             sem):
  wid = lax.axis_index("s") * NUM_CORES + lax.axis_index("c")
  base = wid * BPW

  pltpu.async_copy(pair_hbm.at[pl.ds(base, BPW), :], pairs_v, sem).wait()
  lane = lax.iota(jnp.int32, 16)
  zero16 = jnp.zeros((16,), jnp.int32)
  one16 = jnp.full((16,), 1, jnp.int32)
  for g in range(GROUPS):
    sl = pl.ds(g * LANES, LANES)
    rows = g * LANES + lane
    idx0_v[sl] = plsc.load_gather(pairs_v, [rows, zero16])
    idx1_v[sl] = plsc.load_gather(pairs_v, [rows, one16]) + NB_EMBEDDINGS

  copies = []
  for c in range(NCHUNKS):
    sl = pl.ds(c * CHUNK, CHUNK)
    copies.append(pltpu.async_copy(
        table_hbm.at[idx0_v.at[sl]], rows0_v.at[sl], sem))
    copies.append(pltpu.async_copy(
        table_hbm.at[idx1_v.at[sl]], rows1_v.at[sl], sem))
    copies.append(pltpu.async_copy(
        bias_hbm.at[idx0_v.at[sl]], b0_v.at[sl], sem))
    copies.append(pltpu.async_copy(
        bias_hbm.at[idx1_v.at[sl]], b1_v.at[sl], sem))
  for cp in copies:
    cp.wait()

  def group_body(g, _):
    row_ids = g * LANES + lane

    acc = jnp.zeros((16,), jnp.float32)
    for d in range(EMB_DIM):
      col = jnp.full((16,), d, jnp.int32)
      a = plsc.load_gather(rows0_v, [row_ids, col])
      b = plsc.load_gather(rows1_v, [row_ids, col])
      acc = acc + a * b

    res = acc + b0_v[pl.ds(g * LANES, LANES)] + b1_v[pl.ds(g * LANES, LANES)]
    out_v[pl.ds(g * LANES, LANES)] = res
    return 0

  lax.fori_loop(0, GROUPS, group_body, 0)

  pltpu.sync_copy(out_v, out_hbm.at[pl.ds(base, BPW)])


@jax.jit
def _run(pair, table, bias):
  mesh = plsc.VectorSubcoreMesh(core_axis_name="c", subcore_axis_name="s")
  f = pl.kernel(
      _sc_body,
      out_type=jax.ShapeDtypeStruct((BATCH,), jnp.float32),
      mesh=mesh,
      scratch_types=[
          pltpu.VMEM((BPW, 2), jnp.int32),
          pltpu.VMEM((BPW,), jnp.int32),
          pltpu.VMEM((BPW,), jnp.int32),
          pltpu.VMEM((BPW, EMB_DIM), jnp.float32),
          pltpu.VMEM((BPW, EMB_DIM), jnp.float32),
          pltpu.VMEM((BPW,), jnp.float32),
          pltpu.VMEM((BPW,), jnp.float32),
          pltpu.VMEM((BPW,), jnp.float32),
          pltpu.SemaphoreType.DMA,
      ],
      compiler_params=pltpu.CompilerParams(
          needs_layout_passes=False, use_tc_tiling_on_sc=False),
  )
  return f(pair, table, bias)


def kernel(pair, embedding_table, bias_table):
  bias_flat = bias_table.reshape(-1)
  sim = _run(pair, embedding_table, bias_flat)
  return sim.reshape(BATCH, 1)


# speedup vs baseline: 1.1607x; 1.1607x over previous
import jax
import jax.numpy as jnp
from jax import lax
from jax.experimental import pallas as pl
from jax.experimental.pallas import tpu as pltpu
from jax.experimental.pallas import tpu_sc as plsc

NB_EMBEDDINGS = 100000
NROWS = 2 * NB_EMBEDDINGS
EMB_DIM = 64
BATCH = 16384

NUM_CORES = 2
NUM_SUBCORES = 16
LANES = 16
NUM_WORKERS = NUM_CORES * NUM_SUBCORES
BPW = BATCH // NUM_WORKERS
CHUNK = 128
NCHUNKS = BPW // CHUNK
GROUPS = BPW // LANES


def _sc_body(pair_hbm, table_hbm, bias_hbm, out_hbm,
             pairs_v, idx0_v, idx1_v, rows0_v, rows1_v, b0_v, b1_v, out_v,
             prod_v, sem):
  wid = lax.axis_index("s") * NUM_CORES + lax.axis_index("c")
  base = wid * BPW

  pltpu.async_copy(pair_hbm.at[pl.ds(base, BPW), :], pairs_v, sem).wait()
  lane = lax.iota(jnp.int32, 16)
  zero16 = jnp.zeros((16,), jnp.int32)
  one16 = jnp.full((16,), 1, jnp.int32)
  for g in range(GROUPS):
    sl = pl.ds(g * LANES, LANES)
    rows = g * LANES + lane
    idx0_v[sl] = plsc.load_gather(pairs_v, [rows, zero16])
    idx1_v[sl] = plsc.load_gather(pairs_v, [rows, one16]) + NB_EMBEDDINGS

  copies = []
  for c in range(NCHUNKS):
    sl = pl.ds(c * CHUNK, CHUNK)
    copies.append(pltpu.async_copy(
        table_hbm.at[idx0_v.at[sl]], rows0_v.at[sl], sem))
    copies.append(pltpu.async_copy(
        table_hbm.at[idx1_v.at[sl]], rows1_v.at[sl], sem))
    copies.append(pltpu.async_copy(
        bias_hbm.at[idx0_v.at[sl]], b0_v.at[sl], sem))
    copies.append(pltpu.async_copy(
        bias_hbm.at[idx1_v.at[sl]], b1_v.at[sl], sem))
  for cp in copies:
    cp.wait()

  def pair_body(p, _):
    acc = None
    for k in range(EMB_DIM // LANES):
      a = rows0_v[p, pl.ds(k * LANES, LANES)]
      b = rows1_v[p, pl.ds(k * LANES, LANES)]
      ab = a * b
      acc = ab if acc is None else acc + ab
    prod_v[p, pl.ds(0, LANES)] = acc
    return 0

  lax.fori_loop(0, BPW, pair_body, 0)

  def group_body(g, _):
    row_ids = g * LANES + lane

    acc = jnp.zeros((16,), jnp.float32)
    for d in range(LANES):
      col = jnp.full((16,), d, jnp.int32)
      acc = acc + plsc.load_gather(prod_v, [row_ids, col])

    res = acc + b0_v[pl.ds(g * LANES, LANES)] + b1_v[pl.ds(g * LANES, LANES)]
    out_v[pl.ds(g * LANES, LANES)] = res
    return 0

  lax.fori_loop(0, GROUPS, group_body, 0)

  pltpu.sync_copy(out_v, out_hbm.at[pl.ds(base, BPW)])


@jax.jit
def _run(pair, table, bias):
  mesh = plsc.VectorSubcoreMesh(core_axis_name="c", subcore_axis_name="s")
  f = pl.kernel(
      _sc_body,
      out_type=jax.ShapeDtypeStruct((BATCH,), jnp.float32),
      mesh=mesh,
      scratch_types=[
          pltpu.VMEM((BPW, 2), jnp.int32),
          pltpu.VMEM((BPW,), jnp.int32),
          pltpu.VMEM((BPW,), jnp.int32),
          pltpu.VMEM((BPW, EMB_DIM), jnp.float32),
          pltpu.VMEM((BPW, EMB_DIM), jnp.float32),
          pltpu.VMEM((BPW,), jnp.float32),
          pltpu.VMEM((BPW,), jnp.float32),
          pltpu.VMEM((BPW,), jnp.float32),
          pltpu.VMEM((BPW, 17), jnp.float32),
          pltpu.SemaphoreType.DMA,
      ],
      compiler_params=pltpu.CompilerParams(
          needs_layout_passes=False, use_tc_tiling_on_sc=False),
  )
  return f(pair, table, bias)


def kernel(pair, embedding_table, bias_table):
  bias_flat = bias_table.reshape(-1)
  sim = _run(pair, embedding_table, bias_flat)
  return sim.reshape(BATCH, 1)


# speedup vs baseline: 1.1830x; 1.0192x over previous
import jax
import jax.numpy as jnp
from jax import lax
from jax.experimental import pallas as pl
from jax.experimental.pallas import tpu as pltpu
from jax.experimental.pallas import tpu_sc as plsc

NB_EMBEDDINGS = 100000
NROWS = 2 * NB_EMBEDDINGS
EMB_DIM = 64
BATCH = 16384
TILE_H = 8

NUM_CORES = 2
NUM_SUBCORES = 16
LANES = 16
NUM_WORKERS = NUM_CORES * NUM_SUBCORES
BPW = BATCH // NUM_WORKERS
CHUNK = 128
NCHUNKS = BPW // CHUNK
GROUPS = BPW // LANES
ROUND = 16
NROUNDS = BPW // ROUND


def _sc_body(idx0_hbm, idx1_hbm, tld0_hbm, sub0_hbm, tld1_hbm, sub1_hbm,
             table_hbm, bias_hbm, out_hbm, idx0_v, idx1_v,
             tld0_v, sub0_v, tld1_v, sub1_v, tb0_v, tb1_v,
             b0_v, b1_v, out_v, sem_g, sem_g2, sem_b):
  wid = lax.axis_index("s") * NUM_CORES + lax.axis_index("c")
  base = wid * BPW

  pltpu.sync_copy(tld0_hbm.at[pl.ds(base, BPW)], tld0_v)
  pltpu.sync_copy(tld1_hbm.at[pl.ds(base, BPW)], tld1_v)
  pltpu.sync_copy(sub0_hbm.at[pl.ds(base, BPW)], sub0_v)
  pltpu.sync_copy(sub1_hbm.at[pl.ds(base, BPW)], sub1_v)
  pltpu.sync_copy(idx0_hbm.at[pl.ds(base, BPW)], idx0_v)
  pltpu.sync_copy(idx1_hbm.at[pl.ds(base, BPW)], idx1_v)

  bias_copies = []
  for c in range(NCHUNKS):
    sl = pl.ds(c * CHUNK, CHUNK)
    bias_copies.append(pltpu.async_copy(
        bias_hbm.at[idx0_v.at[sl]], b0_v.at[sl], sem_b))
    bias_copies.append(pltpu.async_copy(
        bias_hbm.at[idx1_v.at[sl]], b1_v.at[sl], sem_b))

  def issue_round(r, par, sem):
    vt0 = tld0_v[pl.ds(r * ROUND, LANES)]
    vt1 = tld1_v[pl.ds(r * ROUND, LANES)]
    for j in range(LANES):
      slot = (par * ROUND + j) * TILE_H
      src0 = pl.ds(pl.multiple_of(vt0[j] * TILE_H, TILE_H), TILE_H)
      src1 = pl.ds(pl.multiple_of(vt1[j] * TILE_H, TILE_H), TILE_H)
      pltpu.async_copy(table_hbm.at[src0, :],
                       tb0_v.at[pl.ds(slot, TILE_H), :], sem)
      pltpu.async_copy(table_hbm.at[src1, :],
                       tb1_v.at[pl.ds(slot, TILE_H), :], sem)

  def wait_round(sem):
    for _ in range(2 * ROUND):
      pltpu.make_async_copy(table_hbm.at[pl.ds(0, TILE_H), :],
                            tb0_v.at[pl.ds(0, TILE_H), :], sem).wait()

  lane = lax.iota(jnp.int32, 16)

  def compute_round(r, par):
    vs0 = sub0_v[pl.ds(r * ROUND, LANES)]
    vs1 = sub1_v[pl.ds(r * ROUND, LANES)]
    acc = jnp.zeros((16,), jnp.float32)
    for j in range(LANES):
      slot = (par * ROUND + j) * TILE_H
      part = None
      for k in range(EMB_DIM // LANES):
        a = tb0_v[slot + vs0[j], pl.ds(k * LANES, LANES)]
        b = tb1_v[slot + vs1[j], pl.ds(k * LANES, LANES)]
        ab = a * b
        part = ab if part is None else part + ab
      s = lax.reduce_sum(part, axes=(0,))
      acc = jnp.where(lane == j, s, acc)
    sl = pl.ds(r * ROUND, LANES)
    out_v[sl] = acc + b0_v[sl] + b1_v[sl]

  issue_round(0, 0, sem_g)

  for cp in bias_copies:
    cp.wait()

  def round_body(r2, _):
    r_even = 2 * r2
    r_odd = r_even + 1

    issue_round(r_odd, 1, sem_g2)
    wait_round(sem_g)
    compute_round(r_even, 0)

    @pl.when(r_even + 2 < NROUNDS)
    def _():
      issue_round(r_even + 2, 0, sem_g)

    wait_round(sem_g2)
    compute_round(r_odd, 1)
    return 0

  lax.fori_loop(0, NROUNDS // 2, round_body, 0)

  pltpu.sync_copy(out_v, out_hbm.at[pl.ds(base, BPW)])


@jax.jit
def _run(idx0, idx1, tld0, sub0, tld1, sub1, table, bias_flat):
  mesh = plsc.VectorSubcoreMesh(core_axis_name="c", subcore_axis_name="s")
  f = pl.kernel(
      _sc_body,
      out_type=jax.ShapeDtypeStruct((BATCH,), jnp.float32),
      mesh=mesh,
      scratch_types=[
          pltpu.VMEM((BPW,), jnp.int32),
          pltpu.VMEM((BPW,), jnp.int32),
          pltpu.VMEM((BPW,), jnp.int32),
          pltpu.VMEM((BPW,), jnp.int32),
          pltpu.VMEM((BPW,), jnp.int32),
          pltpu.VMEM((BPW,), jnp.int32),
          pltpu.VMEM((2 * ROUND * TILE_H, EMB_DIM), jnp.float32),
          pltpu.VMEM((2 * ROUND * TILE_H, EMB_DIM), jnp.float32),
          pltpu.VMEM((BPW,), jnp.float32),
          pltpu.VMEM((BPW,), jnp.float32),
          pltpu.VMEM((BPW,), jnp.float32),
          pltpu.SemaphoreType.DMA,
          pltpu.SemaphoreType.DMA,
          pltpu.SemaphoreType.DMA,
      ],
      compiler_params=pltpu.CompilerParams(
          needs_layout_passes=False, use_tc_tiling_on_sc=True),
  )
  return f(idx0, idx1, tld0, sub0, tld1, sub1, table, bias_flat)


def kernel(pair, embedding_table, bias_table):
  p0 = pair[:, 0].astype(jnp.int32)
  p1 = pair[:, 1].astype(jnp.int32) + NB_EMBEDDINGS
  tld0 = p0 >> 3
  sub0 = p0 & 7
  tld1 = p1 >> 3
  sub1 = p1 & 7
  bias_flat = bias_table.reshape(-1)
  sim = _run(p0, p1, tld0, sub0, tld1, sub1, embedding_table, bias_flat)
  return sim.reshape(BATCH, 1)


# speedup vs baseline: 1.2046x; 1.0183x over previous
import jax
import jax.numpy as jnp
from jax import lax
from jax.experimental import pallas as pl
from jax.experimental.pallas import tpu as pltpu
from jax.experimental.pallas import tpu_sc as plsc

NB_EMBEDDINGS = 100000
NROWS = 2 * NB_EMBEDDINGS
EMB_DIM = 64
BATCH = 16384
TILE_H = 8

NUM_CORES = 2
NUM_SUBCORES = 16
LANES = 16
NUM_WORKERS = NUM_CORES * NUM_SUBCORES
BPW = BATCH // NUM_WORKERS
CHUNK = 128
NCHUNKS = BPW // CHUNK
ROUND = 16
NROUNDS = BPW // ROUND


def _sc_body(idx0_hbm, idx1_hbm, table_hbm, bias_hbm, out_hbm,
             idx0_v, idx1_v, tb0_v, tb1_v, b0_v, b1_v, out_v,
             sem_g, sem_g2, sem_b):
  wid = lax.axis_index("s") * NUM_CORES + lax.axis_index("c")
  base = wid * BPW

  cp0 = pltpu.async_copy(idx0_hbm.at[pl.ds(base, BPW)], idx0_v, sem_b)
  cp1 = pltpu.async_copy(idx1_hbm.at[pl.ds(base, BPW)], idx1_v, sem_b)
  cp0.wait()
  cp1.wait()

  bias_copies = []
  for c in range(NCHUNKS):
    sl = pl.ds(c * CHUNK, CHUNK)
    bias_copies.append(pltpu.async_copy(
        bias_hbm.at[idx0_v.at[sl]], b0_v.at[sl], sem_b))
    bias_copies.append(pltpu.async_copy(
        bias_hbm.at[idx1_v.at[sl]], b1_v.at[sl], sem_b))

  def issue_round(r, par, sem):
    vt0 = idx0_v[pl.ds(r * ROUND, LANES)] >> 3
    vt1 = idx1_v[pl.ds(r * ROUND, LANES)] >> 3
    for j in range(LANES):
      slot = (par * ROUND + j) * TILE_H
      src0 = pl.ds(pl.multiple_of(vt0[j] * TILE_H, TILE_H), TILE_H)
      src1 = pl.ds(pl.multiple_of(vt1[j] * TILE_H, TILE_H), TILE_H)
      pltpu.async_copy(table_hbm.at[src0, :],
                       tb0_v.at[pl.ds(slot, TILE_H), :], sem)
      pltpu.async_copy(table_hbm.at[src1, :],
                       tb1_v.at[pl.ds(slot, TILE_H), :], sem)

  def wait_round(sem):
    for _ in range(2 * ROUND):
      pltpu.make_async_copy(table_hbm.at[pl.ds(0, TILE_H), :],
                            tb0_v.at[pl.ds(0, TILE_H), :], sem).wait()

  lane = lax.iota(jnp.int32, 16)

  def compute_round(r, par):
    vs0 = idx0_v[pl.ds(r * ROUND, LANES)] & 7
    vs1 = idx1_v[pl.ds(r * ROUND, LANES)] & 7
    acc = jnp.zeros((16,), jnp.float32)
    for j in range(LANES):
      slot = (par * ROUND + j) * TILE_H
      part = None
      for k in range(EMB_DIM // LANES):
        a = tb0_v[slot + vs0[j], pl.ds(k * LANES, LANES)]
        b = tb1_v[slot + vs1[j], pl.ds(k * LANES, LANES)]
        ab = a * b
        part = ab if part is None else part + ab
      s = lax.reduce_sum(part, axes=(0,))
      acc = jnp.where(lane == j, s, acc)
    sl = pl.ds(r * ROUND, LANES)
    out_v[sl] = acc + b0_v[sl] + b1_v[sl]

  issue_round(0, 0, sem_g)

  for cp in bias_copies:
    cp.wait()

  def round_body(r2, _):
    r_even = 2 * r2
    r_odd = r_even + 1

    issue_round(r_odd, 1, sem_g2)
    wait_round(sem_g)
    compute_round(r_even, 0)

    @pl.when(r_even + 2 < NROUNDS)
    def _():
      issue_round(r_even + 2, 0, sem_g)

    wait_round(sem_g2)
    compute_round(r_odd, 1)
    return 0

  lax.fori_loop(0, NROUNDS // 2, round_body, 0)

  pltpu.sync_copy(out_v, out_hbm.at[pl.ds(base, BPW)])


@jax.jit
def _run(idx0, idx1, table, bias_flat):
  mesh = plsc.VectorSubcoreMesh(core_axis_name="c", subcore_axis_name="s")
  f = pl.kernel(
      _sc_body,
      out_type=jax.ShapeDtypeStruct((BATCH,), jnp.float32),
      mesh=mesh,
      scratch_types=[
          pltpu.VMEM((BPW,), jnp.int32),
          pltpu.VMEM((BPW,), jnp.int32),
          pltpu.VMEM((2 * ROUND * TILE_H, EMB_DIM), jnp.float32),
          pltpu.VMEM((2 * ROUND * TILE_H, EMB_DIM), jnp.float32),
          pltpu.VMEM((BPW,), jnp.float32),
          pltpu.VMEM((BPW,), jnp.float32),
          pltpu.VMEM((BPW,), jnp.float32),
          pltpu.SemaphoreType.DMA,
          pltpu.SemaphoreType.DMA,
          pltpu.SemaphoreType.DMA,
      ],
      compiler_params=pltpu.CompilerParams(
          needs_layout_passes=False, use_tc_tiling_on_sc=True),
  )
  return f(idx0, idx1, table, bias_flat)


def kernel(pair, embedding_table, bias_table):
  p0 = pair[:, 0].astype(jnp.int32)
  p1 = pair[:, 1].astype(jnp.int32) + NB_EMBEDDINGS
  bias_flat = bias_table.reshape(-1)
  sim = _run(p0, p1, embedding_table, bias_flat)
  return sim.reshape(BATCH, 1)
